# baseline (speedup 1.0000x reference)
import functools

import numpy as np
import jax
import jax.numpy as jnp
from jax.experimental import pallas as pl
from jax.experimental.pallas import tpu as pltpu

_PHYS = {0: 3, 1: 4, 2: 5, 3: 6, 4: 7, 5: 2, 6: 1, 7: 0, 8: 8, 9: 9, 10: 10}
_TILE_ROWS = 256
_TILE_LANES = 128


def _interp_matrix(n: int, m: int) -> np.ndarray:
    j = np.arange(m, dtype=np.int64)
    num = (j + 1) * (n + 1)
    count = (num - 1) // (m + 1)
    ind = np.clip(count - 1, 0, n - 2)
    x_ind = (ind + 1) / (n + 1)
    xnew = (j + 1) / (m + 1)
    dx = 1.0 / (n + 1)
    eps = float(np.finfo(np.float32).eps)
    t = (xnew - x_ind) / (eps + dx)
    a = np.zeros((m, n), dtype=np.float64)
    np.add.at(a, (j, ind), 1.0 - t)
    np.add.at(a, (j, ind + 1), t)
    return a.astype(np.float32)


def _phys_perm(n: int) -> np.ndarray:
    bits = n.bit_length() - 1
    idx = np.arange(n)
    p = np.zeros(n, dtype=np.int64)
    for b in range(bits):
        p |= ((idx >> b) & 1) << _PHYS[b]
    return p


def _build_schedule(bits: int):
    sched = []
    tile_ops = [('s', _PHYS[1], None)]

    def flush():
        nonlocal tile_ops
        if tile_ops:
            sched.append(('tile', tile_ops))
            tile_ops = []

    for a in range(1, bits + 1):
        for b in range(a - 1, -1, -1):
            pd = 1 << _PHYS[b]
            if pd >= _TILE_ROWS:
                flush()
                sched.append(('full', pd))
            else:
                tile_ops.append(('c', pd, None))
        if a < bits:
            b2 = _PHYS[a + 1] if a + 1 < bits else None
            tile_ops.append(('s', _PHYS[a], b2))
    flush()
    return sched


def _single_bit_reshape(x, jp):
    n, s = x.shape
    g = n // (2 * jp)
    xr = x.reshape(g, 2, jp, s)
    a = xr[:, 0]
    b = xr[:, 1]
    mn = jnp.minimum(a, b)[:, None]
    mx = jnp.maximum(a, b)[:, None]
    return jnp.concatenate([mn, mx], axis=1).reshape(n, s)


def _single_bit_roll(x, rloc, jp):
    low = (rloc & jp) == 0
    mn = jnp.minimum(x, jnp.roll(x, -jp, axis=0))
    mx = jnp.maximum(x, jnp.roll(x, jp, axis=0))
    return jnp.where(low, mn, mx)


def _run_tile_ops(t, ops, row_base, rloc):
    lb = _TILE_ROWS.bit_length() - 1

    def bitval(b):
        if b >= lb:
            return bool((row_base >> b) & 1)
        return (rloc & (1 << b)) != 0

    for op in ops:
        kind, x1, x2 = op
        if kind == 'c':
            if x1 >= 8:
                t = _single_bit_reshape(t, x1)
            else:
                t = _single_bit_roll(t, rloc, x1)
        else:
            m1 = bitval(x1)
            m2 = bitval(x2) if x2 is not None else False
            if isinstance(m1, bool) and isinstance(m2, bool):
                if m1 != m2:
                    t = -t
            else:
                if isinstance(m1, bool):
                    flip = jnp.logical_not(m2) if m1 else m2
                elif isinstance(m2, bool):
                    flip = jnp.logical_not(m1) if m2 else m1
                else:
                    flip = m1 != m2
                t = jnp.where(flip, -t, t)
    return t


def _tile_pass(y, ops):
    n, s = y.shape
    tr, tl = _TILE_ROWS, _TILE_LANES
    rloc = jax.lax.broadcasted_iota(jnp.int32, (tr, 1), 0)
    rows = []
    for r in range(n // tr):
        cols = []
        for l in range(s // tl):
            t = y[r * tr:(r + 1) * tr, l * tl:(l + 1) * tl]
            cols.append(_run_tile_ops(t, ops, r * tr, rloc))
        rows.append(jnp.concatenate(cols, axis=1))
    return jnp.concatenate(rows, axis=0)


def _bitonic_sort_permuted(x):
    n = x.shape[0]
    bits = n.bit_length() - 1
    for kind, arg in _build_schedule(bits):
        if kind == 'full':
            x = _single_bit_reshape(x, arg)
        else:
            x = _tile_pass(x, arg)
    return x


def _body(x_ref, th_ref, ref_ref, w_ref, a_ref, out_ref):
    th = th_ref[...]
    wn = th / jnp.sqrt(jnp.sum(th * th, axis=1, keepdims=True))
    proj = jax.lax.dot_general(
        x_ref[0], wn, (((1,), (1,)), ((), ())),
        precision=jax.lax.Precision.HIGHEST,
        preferred_element_type=jnp.float32)
    srt = _bitonic_sort_permuted(proj)
    w = w_ref[...]
    hi = jax.lax.Precision.HIGHEST
    wa = jax.lax.dot_general(
        w, a_ref[...], (((1,), (0,)), ((), ())),
        precision=hi, preferred_element_type=jnp.float32)
    red = jax.lax.dot_general(
        wa, srt, (((1,), (0,)), ((), ())),
        precision=hi, preferred_element_type=jnp.float32)
    cst = jax.lax.dot_general(
        w, ref_ref[...], (((1,), (0,)), ((), ())),
        precision=hi, preferred_element_type=jnp.float32)
    out_ref[...] = (cst - red)[None]


def kernel(X, theta_v, reference_pts, w):
    b, n, d = X.shape
    s = theta_v.shape[0]
    m = reference_pts.shape[0]
    a_np = _interp_matrix(n, m)
    perm = _phys_perm(n)
    a_perm = np.zeros_like(a_np)
    a_perm[:, perm] = a_np
    a_mat = jnp.asarray(a_perm)

    s_blk = 256
    grid = (b, s // s_blk)

    out3 = pl.pallas_call(
        _body,
        grid=grid,
        in_specs=[
            pl.BlockSpec((1, n, d), lambda i, j: (i, 0, 0)),
            pl.BlockSpec((s_blk, d), lambda i, j: (j, 0)),
            pl.BlockSpec((m, s_blk), lambda i, j: (0, j)),
            pl.BlockSpec((1, m), lambda i, j: (0, 0)),
            pl.BlockSpec((m, n), lambda i, j: (0, 0)),
        ],
        out_specs=pl.BlockSpec((1, 1, s_blk), lambda i, j: (i, 0, j)),
        out_shape=jax.ShapeDtypeStruct((b, 1, s), jnp.float32),
        compiler_params=pltpu.CompilerParams(
            dimension_semantics=("parallel", "parallel"),
        ),
    )(X, theta_v, reference_pts, w, a_mat)
    return out3.reshape(b, s)


# speedup vs baseline: 1.2491x; 1.2491x over previous
import functools

import numpy as np
import jax
import jax.numpy as jnp
from jax.experimental import pallas as pl
from jax.experimental.pallas import tpu as pltpu

_PHYS = {0: 3, 1: 4, 2: 5, 3: 6, 4: 7, 5: 8, 6: 9, 7: 10, 8: 2, 9: 1, 10: 0}
_TILE_ROWS = 256
_TILE_LANES = 128


def _interp_matrix(n: int, m: int) -> np.ndarray:
    j = np.arange(m, dtype=np.int64)
    num = (j + 1) * (n + 1)
    count = (num - 1) // (m + 1)
    ind = np.clip(count - 1, 0, n - 2)
    x_ind = (ind + 1) / (n + 1)
    xnew = (j + 1) / (m + 1)
    dx = 1.0 / (n + 1)
    eps = float(np.finfo(np.float32).eps)
    t = (xnew - x_ind) / (eps + dx)
    a = np.zeros((m, n), dtype=np.float64)
    np.add.at(a, (j, ind), 1.0 - t)
    np.add.at(a, (j, ind + 1), t)
    return a.astype(np.float32)


def _phys_perm(n: int) -> np.ndarray:
    bits = n.bit_length() - 1
    idx = np.arange(n)
    p = np.zeros(n, dtype=np.int64)
    for b in range(bits):
        p |= ((idx >> b) & 1) << _PHYS[b]
    return p


def _build_schedule(bits: int):
    sched = []
    tile_ops = [('s', _PHYS[1], None)]

    def flush():
        nonlocal tile_ops
        if tile_ops:
            sched.append(('tile', tile_ops))
            tile_ops = []

    for a in range(1, bits + 1):
        for b in range(a - 1, -1, -1):
            pd = 1 << _PHYS[b]
            if pd >= _TILE_ROWS:
                flush()
                sched.append(('full', pd))
            else:
                tile_ops.append(('c', pd, None))
        if a < bits:
            b2 = _PHYS[a + 1] if a + 1 < bits else None
            tile_ops.append(('s', _PHYS[a], b2))
    flush()
    return sched


def _single_bit_reshape(x, jp):
    n, s = x.shape
    g = n // (2 * jp)
    xr = x.reshape(g, 2, jp, s)
    a = xr[:, 0]
    b = xr[:, 1]
    mn = jnp.minimum(a, b)[:, None]
    mx = jnp.maximum(a, b)[:, None]
    return jnp.concatenate([mn, mx], axis=1).reshape(n, s)


def _single_bit_roll(x, rloc, jp):
    low = (rloc & jp) == 0
    mn = jnp.minimum(x, jnp.roll(x, -jp, axis=0))
    mx = jnp.maximum(x, jnp.roll(x, jp, axis=0))
    return jnp.where(low, mn, mx)


def _run_tile_ops(t, ops, row_base, rloc):
    lb = _TILE_ROWS.bit_length() - 1

    def bitval(b):
        if b >= lb:
            return bool((row_base >> b) & 1)
        return (rloc & (1 << b)) != 0

    for op in ops:
        kind, x1, x2 = op
        if kind == 'c':
            if x1 >= 8:
                t = _single_bit_reshape(t, x1)
            else:
                t = _single_bit_roll(t, rloc, x1)
        else:
            m1 = bitval(x1)
            m2 = bitval(x2) if x2 is not None else False
            if isinstance(m1, bool) and isinstance(m2, bool):
                if m1 != m2:
                    t = -t
            else:
                if isinstance(m1, bool):
                    flip = jnp.logical_not(m2) if m1 else m2
                elif isinstance(m2, bool):
                    flip = jnp.logical_not(m1) if m2 else m1
                else:
                    flip = m1 != m2
                t = jnp.where(flip, -t, t)
    return t


def _tile_pass(y, ops):
    n, s = y.shape
    tr, tl = _TILE_ROWS, _TILE_LANES
    rloc = jax.lax.broadcasted_iota(jnp.int32, (tr, 1), 0)
    rows = []
    for r in range(n // tr):
        cols = []
        for l in range(s // tl):
            t = y[r * tr:(r + 1) * tr, l * tl:(l + 1) * tl]
            cols.append(_run_tile_ops(t, ops, r * tr, rloc))
        rows.append(jnp.concatenate(cols, axis=1))
    return jnp.concatenate(rows, axis=0)


def _bitonic_sort_permuted(x):
    n = x.shape[0]
    bits = n.bit_length() - 1
    for kind, arg in _build_schedule(bits):
        if kind == 'full':
            x = _single_bit_reshape(x, arg)
        else:
            x = _tile_pass(x, arg)
    return x


def _body(x_ref, th_ref, ref_ref, w_ref, a_ref, out_ref):
    th = th_ref[...]
    wn = th / jnp.sqrt(jnp.sum(th * th, axis=1, keepdims=True))
    proj = jax.lax.dot_general(
        x_ref[0], wn, (((1,), (1,)), ((), ())),
        precision=jax.lax.Precision.HIGHEST,
        preferred_element_type=jnp.float32)
    srt = _bitonic_sort_permuted(proj)
    w = w_ref[...]
    hi = jax.lax.Precision.HIGHEST
    wa = jax.lax.dot_general(
        w, a_ref[...], (((1,), (0,)), ((), ())),
        precision=hi, preferred_element_type=jnp.float32)
    red = jax.lax.dot_general(
        wa, srt, (((1,), (0,)), ((), ())),
        precision=hi, preferred_element_type=jnp.float32)
    cst = jax.lax.dot_general(
        w, ref_ref[...], (((1,), (0,)), ((), ())),
        precision=hi, preferred_element_type=jnp.float32)
    out_ref[...] = (cst - red)[None]


def kernel(X, theta_v, reference_pts, w):
    b, n, d = X.shape
    s = theta_v.shape[0]
    m = reference_pts.shape[0]
    a_np = _interp_matrix(n, m)
    perm = _phys_perm(n)
    a_perm = np.zeros_like(a_np)
    a_perm[:, perm] = a_np
    a_mat = jnp.asarray(a_perm)

    s_blk = 256
    grid = (b, s // s_blk)

    out3 = pl.pallas_call(
        _body,
        grid=grid,
        in_specs=[
            pl.BlockSpec((1, n, d), lambda i, j: (i, 0, 0)),
            pl.BlockSpec((s_blk, d), lambda i, j: (j, 0)),
            pl.BlockSpec((m, s_blk), lambda i, j: (0, j)),
            pl.BlockSpec((1, m), lambda i, j: (0, 0)),
            pl.BlockSpec((m, n), lambda i, j: (0, 0)),
        ],
        out_specs=pl.BlockSpec((1, 1, s_blk), lambda i, j: (i, 0, j)),
        out_shape=jax.ShapeDtypeStruct((b, 1, s), jnp.float32),
        compiler_params=pltpu.CompilerParams(
            dimension_semantics=("parallel", "parallel"),
        ),
    )(X, theta_v, reference_pts, w, a_mat)
    return out3.reshape(b, s)


# speedup vs baseline: 1.3935x; 1.1156x over previous
import functools

import numpy as np
import jax
import jax.numpy as jnp
from jax.experimental import pallas as pl
from jax.experimental.pallas import tpu as pltpu

_PHYS = {0: 3, 1: 4, 2: 5, 3: 6, 4: 7, 5: 8, 6: 9, 7: 10, 8: 2, 9: 1, 10: 0}
_TILE_ROWS = 256
_TILE_LANES = 128


def _interp_matrix(n: int, m: int) -> np.ndarray:
    j = np.arange(m, dtype=np.int64)
    num = (j + 1) * (n + 1)
    count = (num - 1) // (m + 1)
    ind = np.clip(count - 1, 0, n - 2)
    x_ind = (ind + 1) / (n + 1)
    xnew = (j + 1) / (m + 1)
    dx = 1.0 / (n + 1)
    eps = float(np.finfo(np.float32).eps)
    t = (xnew - x_ind) / (eps + dx)
    a = np.zeros((m, n), dtype=np.float64)
    np.add.at(a, (j, ind), 1.0 - t)
    np.add.at(a, (j, ind + 1), t)
    return a.astype(np.float32)


def _phys_perm(n: int) -> np.ndarray:
    bits = n.bit_length() - 1
    idx = np.arange(n)
    p = np.zeros(n, dtype=np.int64)
    for b in range(bits):
        p |= ((idx >> b) & 1) << _PHYS[b]
    return p


def _build_schedule(bits: int):
    sched = []
    tile_ops = [('s', _PHYS[1], None)]

    def flush():
        nonlocal tile_ops
        if tile_ops:
            sched.append(('tile', tile_ops))
            tile_ops = []

    for a in range(1, bits + 1):
        for b in range(a - 1, -1, -1):
            pd = 1 << _PHYS[b]
            if pd >= _TILE_ROWS:
                flush()
                sched.append(('full', pd))
            else:
                tile_ops.append(('c', pd, None))
        if a < bits:
            b2 = _PHYS[a + 1] if a + 1 < bits else None
            tile_ops.append(('s', _PHYS[a], b2))
    flush()
    return sched


def _single_bit_reshape(x, jp):
    n, s = x.shape
    g = n // (2 * jp)
    xr = x.reshape(g, 2, jp, s)
    a = xr[:, 0]
    b = xr[:, 1]
    mn = jnp.minimum(a, b)[:, None]
    mx = jnp.maximum(a, b)[:, None]
    return jnp.concatenate([mn, mx], axis=1).reshape(n, s)


def _single_bit_roll(x, rloc, jp):
    low = (rloc & jp) == 0
    mn = jnp.minimum(x, jnp.roll(x, -jp, axis=0))
    mx = jnp.maximum(x, jnp.roll(x, jp, axis=0))
    return jnp.where(low, mn, mx)


def _run_tile_ops(t, ops, row_base, rloc):
    lb = _TILE_ROWS.bit_length() - 1

    def bitval(b):
        if b >= lb:
            return bool((row_base >> b) & 1)
        return (rloc & (1 << b)) != 0

    for op in ops:
        kind, x1, x2 = op
        if kind == 'c':
            if x1 >= 8:
                t = _single_bit_reshape(t, x1)
            else:
                t = _single_bit_roll(t, rloc, x1)
        else:
            m1 = bitval(x1)
            m2 = bitval(x2) if x2 is not None else False
            if isinstance(m1, bool) and isinstance(m2, bool):
                if m1 != m2:
                    t = -t
            else:
                if isinstance(m1, bool):
                    flip = jnp.logical_not(m2) if m1 else m2
                elif isinstance(m2, bool):
                    flip = jnp.logical_not(m1) if m2 else m1
                else:
                    flip = m1 != m2
                t = jnp.where(flip, -t, t)
    return t


def _tile_pass(y, ops):
    n, s = y.shape
    tr, tl = _TILE_ROWS, _TILE_LANES
    rloc = jax.lax.broadcasted_iota(jnp.int32, (tr, 1), 0)
    rows = []
    for r in range(n // tr):
        cols = []
        for l in range(s // tl):
            t = y[r * tr:(r + 1) * tr, l * tl:(l + 1) * tl]
            cols.append(_run_tile_ops(t, ops, r * tr, rloc))
        rows.append(jnp.concatenate(cols, axis=1))
    return jnp.concatenate(rows, axis=0)


def _bitonic_sort_permuted(x):
    n = x.shape[0]
    bits = n.bit_length() - 1
    for kind, arg in _build_schedule(bits):
        if kind == 'full':
            x = _single_bit_reshape(x, arg)
        else:
            x = _tile_pass(x, arg)
    return x


def _body(x_ref, th_ref, ref_ref, w_ref, a_ref, out_ref):
    th = th_ref[...]
    wn = th / jnp.sqrt(jnp.sum(th * th, axis=1, keepdims=True))
    proj = jax.lax.dot_general(
        x_ref[0], wn, (((1,), (1,)), ((), ())),
        precision=jax.lax.Precision.HIGHEST,
        preferred_element_type=jnp.float32)
    srt = _bitonic_sort_permuted(proj)
    w = w_ref[...]
    hi = jax.lax.Precision.HIGHEST
    wa = jax.lax.dot_general(
        w, a_ref[...], (((1,), (0,)), ((), ())),
        precision=hi, preferred_element_type=jnp.float32)
    red = jax.lax.dot_general(
        wa, srt, (((1,), (0,)), ((), ())),
        precision=hi, preferred_element_type=jnp.float32)
    cst = jax.lax.dot_general(
        w, ref_ref[...], (((1,), (0,)), ((), ())),
        precision=hi, preferred_element_type=jnp.float32)
    out_ref[...] = (cst - red)[None]


def kernel(X, theta_v, reference_pts, w):
    b, n, d = X.shape
    s = theta_v.shape[0]
    m = reference_pts.shape[0]
    a_np = _interp_matrix(n, m)
    perm = _phys_perm(n)
    a_perm = np.zeros_like(a_np)
    a_perm[:, perm] = a_np
    a_mat = jnp.asarray(a_perm)

    s_blk = 512
    grid = (b, s // s_blk)

    out3 = pl.pallas_call(
        _body,
        grid=grid,
        in_specs=[
            pl.BlockSpec((1, n, d), lambda i, j: (i, 0, 0)),
            pl.BlockSpec((s_blk, d), lambda i, j: (j, 0)),
            pl.BlockSpec((m, s_blk), lambda i, j: (0, j)),
            pl.BlockSpec((1, m), lambda i, j: (0, 0)),
            pl.BlockSpec((m, n), lambda i, j: (0, 0)),
        ],
        out_specs=pl.BlockSpec((1, 1, s_blk), lambda i, j: (i, 0, j)),
        out_shape=jax.ShapeDtypeStruct((b, 1, s), jnp.float32),
        compiler_params=pltpu.CompilerParams(
            dimension_semantics=("parallel", "parallel"),
        ),
    )(X, theta_v, reference_pts, w, a_mat)
    return out3.reshape(b, s)
